# baseline (speedup 1.0000x reference)
import functools

import jax
import jax.numpy as jnp
from jax import lax
from jax.experimental import pallas as pl
from jax.experimental.pallas import tpu as pltpu
from jax.experimental.pallas import tpu_sc as plsc

SEQ_LEN = 2048
D_MODEL = 1024


def _make_sc_broadcast(n: int):
    info = plsc.get_sparse_core_info()
    num_workers = info.num_cores * info.num_subcores
    rows_per_w = SEQ_LEN // num_workers
    mesh = plsc.VectorSubcoreMesh(core_axis_name="c", subcore_axis_name="s")

    @functools.partial(
        pl.kernel,
        mesh=mesh,
        out_type=jax.ShapeDtypeStruct((SEQ_LEN, n * D_MODEL), jnp.float32),
        scratch_types=[
            pltpu.VMEM((rows_per_w, D_MODEL), jnp.float32),
            pltpu.SemaphoreType.DMA,
            pltpu.SemaphoreType.DMA,
        ],
    )
    def sc_broadcast(pe_hbm, out_hbm, buf, sem_in, sem_out):
        wid = lax.axis_index("s") * info.num_cores + lax.axis_index("c")
        base = wid * rows_per_w
        pltpu.async_copy(pe_hbm.at[pl.ds(base, rows_per_w)], buf, sem_in).wait()
        copies = [
            pltpu.async_copy(
                buf,
                out_hbm.at[pl.ds(base, rows_per_w), pl.ds(j * D_MODEL, D_MODEL)],
                sem_out,
            )
            for j in range(n)
        ]
        for cp in copies:
            cp.wait()

    return sc_broadcast


def kernel(z, pe):
    n = z.shape[1]
    out_flat = _make_sc_broadcast(n)(pe)
    return out_flat.reshape(SEQ_LEN, n, D_MODEL)


# speedup vs baseline: 2.2409x; 2.2409x over previous
import functools

import jax
import jax.numpy as jnp
from jax import lax
from jax.experimental import pallas as pl
from jax.experimental.pallas import tpu as pltpu
from jax.experimental.pallas import tpu_sc as plsc

SEQ_LEN = 2048
D_MODEL = 1024


def _make_sc_broadcast(n: int):
    info = plsc.get_sparse_core_info()
    num_workers = info.num_cores * info.num_subcores
    rows_per_w = SEQ_LEN // num_workers
    mesh = plsc.VectorSubcoreMesh(core_axis_name="c", subcore_axis_name="s")

    @functools.partial(
        pl.kernel,
        mesh=mesh,
        out_type=jax.ShapeDtypeStruct((SEQ_LEN, n, D_MODEL), jnp.float32),
        scratch_types=[
            pltpu.VMEM((rows_per_w, D_MODEL), jnp.float32),
            pltpu.SemaphoreType.DMA,
            pltpu.SemaphoreType.DMA,
        ],
    )
    def sc_broadcast(pe_hbm, out_hbm, buf, sem_in, sem_out):
        wid = lax.axis_index("s") * info.num_cores + lax.axis_index("c")
        base = wid * rows_per_w
        pltpu.async_copy(pe_hbm.at[pl.ds(base, rows_per_w)], buf, sem_in).wait()
        copies = [
            pltpu.async_copy(
                buf,
                out_hbm.at[pl.ds(base, rows_per_w), j],
                sem_out,
            )
            for j in range(n)
        ]
        for cp in copies:
            cp.wait()

    return sc_broadcast


def kernel(z, pe):
    n = z.shape[1]
    return _make_sc_broadcast(n)(pe)
